# baseline (speedup 1.0000x reference)
import functools

import jax
import jax.numpy as jnp
from jax import lax
from jax.experimental import pallas as pl
from jax.experimental.pallas import tpu as pltpu
from jax.experimental.pallas import tpu_sc as plsc

N = 10000
D = 128
E = 320000
NC = 2
NS = 16
C = 88
NB = 4
NI = 2 * NB
U = NI
NCH = 228
EPT = NCH * C
ACC_ROWS = 10016
WR = 624


def _mm_body(x_ref, w_ref, o_ref):
    o_ref[...] = jnp.dot(x_ref[...], w_ref[...],
                         preferred_element_type=jnp.float32)


def _matmul(x, w):
    m = x.shape[0]
    bm = 1000
    return pl.pallas_call(
        _mm_body,
        grid=(m // bm,),
        in_specs=[pl.BlockSpec((bm, D), lambda i: (i, 0)),
                  pl.BlockSpec((D, D), lambda i: (0, 0))],
        out_specs=pl.BlockSpec((bm, D), lambda i: (i, 0)),
        out_shape=jax.ShapeDtypeStruct((m, D), jnp.float32),
    )(x, w)


def _prep_edges(edge_index):
    pad = NS * EPT - E
    src = jnp.concatenate(
        [edge_index[0].astype(jnp.int32),
         jnp.arange(pad, dtype=jnp.int32) % N]).reshape(NS, NCH, C)
    dst = jnp.concatenate(
        [edge_index[1].astype(jnp.int32),
         N + (jnp.arange(pad, dtype=jnp.int32) % NS)]).reshape(NS, NCH, C)
    packed = jnp.stack([src, dst], axis=2)
    extra = jnp.zeros((NS, NB, 2, C), jnp.int32)
    return jnp.concatenate([packed, extra], axis=1)


def _sc_body(h0, h1, idx0, idx1, zrows,
             out0, out1,
             acc, bufs, ibs, sgs, scs, sis):
    c = lax.axis_index("c")
    s = lax.axis_index("s")

    @pl.when(s < NS - 1)
    def _():
        pltpu.sync_copy(zrows.at[pl.ds(0, WR)], acc.at[pl.ds(s * WR, WR)])

    @pl.when(s == NS - 1)
    def _():
        last = (NS - 1) * WR
        pltpu.sync_copy(zrows, acc.at[pl.ds(last, ACC_ROWS - last)])

    plsc.subcore_barrier()

    def run(h, idx, out):
        def idx_start(k, q):
            pltpu.make_async_copy(idx.at[s, k], ibs[q], sis[q]).start()

        def idx_wait(k, q):
            pltpu.make_async_copy(idx.at[s, k], ibs[q], sis[q]).wait()

        def gather_start(p, q):
            pltpu.make_async_copy(h.at[ibs[q].at[0]], bufs[p], sgs[p]).start()

        def gather_wait(p, q):
            pltpu.make_async_copy(h.at[ibs[q].at[0]], bufs[p], sgs[p]).wait()

        def scatter_start(p, q):
            pltpu.async_copy(bufs[p], acc.at[ibs[q].at[1]], scs[p], add=True)

        def scatter_wait(p, q):
            pltpu.make_async_copy(bufs[p], acc.at[ibs[q].at[1]], scs[p]).wait()

        for k in range(NB):
            idx_start(k, k)
        for j in range(NB):
            idx_wait(j, j)
            gather_start(j, j)
            idx_start(j + NB, (j + NB) % NI)
            if j > 0:
                gather_wait(j - 1, j - 1)
                scatter_start(j - 1, j - 1)

        def body(g, carry):
            for r in range(U):
                j = NB + U * g + r
                p, q = (NB + r) % NB, (NB + r) % NI
                p1, q1 = (NB + r - 1) % NB, (NB + r - 1) % NI
                q3 = (NB + r + NB) % NI
                scatter_wait(p, q3)
                idx_wait(j, q)
                gather_start(p, q)
                idx_start(j + NB, q3)
                gather_wait(p1, q1)
                scatter_start(p1, q1)
            return carry

        lax.fori_loop(0, (NCH - NB) // U, body, 0)

        pL, qL = (NCH - 1) % NB, (NCH - 1) % NI
        gather_wait(pL, qL)
        scatter_start(pL, qL)
        for k in range(NCH - NB, NCH):
            scatter_wait(k % NB, k % NI)
        for k in range(NCH, NCH + NB):
            idx_wait(k, k % NI)

        plsc.subcore_barrier()

        @pl.when(s < NS - 1)
        def _():
            pltpu.sync_copy(acc.at[pl.ds(s * WR, WR)],
                            out.at[pl.ds(s * WR, WR)])

        @pl.when(s == NS - 1)
        def _():
            last = (NS - 1) * WR
            pltpu.sync_copy(acc.at[pl.ds(last, N - last)],
                            out.at[pl.ds(last, N - last)])

    @pl.when(c == 0)
    def _():
        run(h0, idx0, out0)

    @pl.when(c == 1)
    def _():
        run(h1, idx1, out1)


@functools.partial(
    pl.kernel,
    out_type=[jax.ShapeDtypeStruct((N, D), jnp.float32),
              jax.ShapeDtypeStruct((N, D), jnp.float32)],
    mesh=plsc.VectorSubcoreMesh(core_axis_name="c", subcore_axis_name="s",
                                num_cores=NC, num_subcores=NS),
    scratch_types=[
        pltpu.VMEM_SHARED((ACC_ROWS, D), jnp.float32),
        [pltpu.VMEM((C, D), jnp.float32) for _ in range(NB)],
        [pltpu.VMEM((2, C), jnp.int32) for _ in range(NI)],
        [pltpu.SemaphoreType.DMA for _ in range(NB)],
        [pltpu.SemaphoreType.DMA for _ in range(NB)],
        [pltpu.SemaphoreType.DMA for _ in range(NI)],
    ],
)
def _sc_conv(h0, h1, idx0, idx1, zrows, out0, out1,
             acc, bufs, ibs, sgs, scs, sis):
    _sc_body(h0, h1, idx0, idx1, zrows, out0, out1,
             acc, bufs, ibs, sgs, scs, sis)


def kernel(x_user, x_item, W_u2i, W_i2u, edge_index_u2i, edge_index_i2u):
    h_u = _matmul(x_user, W_u2i)
    h_i = _matmul(x_item, W_i2u)
    idx_u = _prep_edges(edge_index_i2u)
    idx_i = _prep_edges(edge_index_u2i)
    zrows = jnp.zeros((ACC_ROWS - (NS - 1) * WR, D), jnp.float32)
    agg_user, agg_item = _sc_conv(h_i, h_u, idx_u, idx_i, zrows)
    return (agg_user, agg_item)


# speedup vs baseline: 1.0690x; 1.0690x over previous
import functools

import jax
import jax.numpy as jnp
from jax import lax
from jax.experimental import pallas as pl
from jax.experimental.pallas import tpu as pltpu
from jax.experimental.pallas import tpu_sc as plsc

N = 10000
D = 128
E = 320000
NC = 2
NS = 16
C = 92
M = 2
CP = M * C
NB = 2
NI = 2 * NB
U = NI
NCH = 110
EPT = NCH * CP
ACC_ROWS = 10016
WR = 624


def _mm_body(x_ref, w_ref, o_ref):
    o_ref[...] = jnp.dot(x_ref[...], w_ref[...],
                         preferred_element_type=jnp.float32)


def _matmul(x, w):
    m = x.shape[0]
    bm = 1000
    return pl.pallas_call(
        _mm_body,
        grid=(m // bm,),
        in_specs=[pl.BlockSpec((bm, D), lambda i: (i, 0)),
                  pl.BlockSpec((D, D), lambda i: (0, 0))],
        out_specs=pl.BlockSpec((bm, D), lambda i: (i, 0)),
        out_shape=jax.ShapeDtypeStruct((m, D), jnp.float32),
    )(x, w)


def _prep_edges(edge_index):
    pad = NS * EPT - E
    src = jnp.concatenate(
        [edge_index[0].astype(jnp.int32),
         jnp.arange(pad, dtype=jnp.int32) % N]).reshape(NS, NCH, M, C)
    dst = jnp.concatenate(
        [edge_index[1].astype(jnp.int32),
         N + (jnp.arange(pad, dtype=jnp.int32) % NS)]).reshape(NS, NCH, M, C)
    packed = jnp.concatenate([src, dst], axis=2)
    extra = jnp.zeros((NS, NB, 2 * M, C), jnp.int32)
    return jnp.concatenate([packed, extra], axis=1)


def _sc_body(h0, h1, idx0, idx1, zrows,
             out0, out1,
             acc, bufs, ibs, sgs, scs, sis):
    c = lax.axis_index("c")
    s = lax.axis_index("s")

    @pl.when(s < NS - 1)
    def _():
        pltpu.sync_copy(zrows.at[pl.ds(0, WR)], acc.at[pl.ds(s * WR, WR)])

    @pl.when(s == NS - 1)
    def _():
        last = (NS - 1) * WR
        pltpu.sync_copy(zrows, acc.at[pl.ds(last, ACC_ROWS - last)])

    plsc.subcore_barrier()

    def run(h, idx, out):
        def idx_start(k, q):
            pltpu.make_async_copy(idx.at[s, k], ibs[q], sis[q]).start()

        def idx_wait(k, q):
            pltpu.make_async_copy(idx.at[s, k], ibs[q], sis[q]).wait()

        def gather_start(p, q):
            for m in range(M):
                pltpu.make_async_copy(h.at[ibs[q].at[m]],
                                      bufs[p].at[pl.ds(m * C, C)],
                                      sgs[p]).start()

        def gather_wait(p, q):
            for m in range(M):
                pltpu.make_async_copy(h.at[ibs[q].at[m]],
                                      bufs[p].at[pl.ds(m * C, C)],
                                      sgs[p]).wait()

        def scatter_start(p, q):
            for m in range(M):
                pltpu.async_copy(bufs[p].at[pl.ds(m * C, C)],
                                 acc.at[ibs[q].at[M + m]], scs[p], add=True)

        def scatter_wait(p, q):
            for m in range(M):
                pltpu.make_async_copy(bufs[p].at[pl.ds(m * C, C)],
                                      acc.at[ibs[q].at[M + m]], scs[p]).wait()

        for k in range(NB):
            idx_start(k, k)
        for j in range(NB):
            idx_wait(j, j)
            gather_start(j, j)
            idx_start(j + NB, (j + NB) % NI)
            if j > 0:
                gather_wait(j - 1, j - 1)
                scatter_start(j - 1, j - 1)

        def body(g, carry):
            for r in range(U):
                j = NB + U * g + r
                p, q = (NB + r) % NB, (NB + r) % NI
                p1, q1 = (NB + r - 1) % NB, (NB + r - 1) % NI
                q3 = (NB + r + NB) % NI
                scatter_wait(p, q3)
                idx_wait(j, q)
                gather_start(p, q)
                idx_start(j + NB, q3)
                gather_wait(p1, q1)
                scatter_start(p1, q1)
            return carry

        lax.fori_loop(0, (NCH - NB) // U, body, 0)

        pL, qL = (NCH - 1) % NB, (NCH - 1) % NI
        gather_wait(pL, qL)
        scatter_start(pL, qL)
        for k in range(NCH - NB, NCH):
            scatter_wait(k % NB, k % NI)
        for k in range(NCH, NCH + NB):
            idx_wait(k, k % NI)

        plsc.subcore_barrier()

        @pl.when(s < NS - 1)
        def _():
            pltpu.sync_copy(acc.at[pl.ds(s * WR, WR)],
                            out.at[pl.ds(s * WR, WR)])

        @pl.when(s == NS - 1)
        def _():
            last = (NS - 1) * WR
            pltpu.sync_copy(acc.at[pl.ds(last, N - last)],
                            out.at[pl.ds(last, N - last)])

    @pl.when(c == 0)
    def _():
        run(h0, idx0, out0)

    @pl.when(c == 1)
    def _():
        run(h1, idx1, out1)


@functools.partial(
    pl.kernel,
    out_type=[jax.ShapeDtypeStruct((N, D), jnp.float32),
              jax.ShapeDtypeStruct((N, D), jnp.float32)],
    mesh=plsc.VectorSubcoreMesh(core_axis_name="c", subcore_axis_name="s",
                                num_cores=NC, num_subcores=NS),
    scratch_types=[
        pltpu.VMEM_SHARED((ACC_ROWS, D), jnp.float32),
        [pltpu.VMEM((CP, D), jnp.float32) for _ in range(NB)],
        [pltpu.VMEM((2 * M, C), jnp.int32) for _ in range(NI)],
        [pltpu.SemaphoreType.DMA for _ in range(NB)],
        [pltpu.SemaphoreType.DMA for _ in range(NB)],
        [pltpu.SemaphoreType.DMA for _ in range(NI)],
    ],
)
def _sc_conv(h0, h1, idx0, idx1, zrows, out0, out1,
             acc, bufs, ibs, sgs, scs, sis):
    _sc_body(h0, h1, idx0, idx1, zrows, out0, out1,
             acc, bufs, ibs, sgs, scs, sis)


def kernel(x_user, x_item, W_u2i, W_i2u, edge_index_u2i, edge_index_i2u):
    h_u = _matmul(x_user, W_u2i)
    h_i = _matmul(x_item, W_i2u)
    idx_u = _prep_edges(edge_index_i2u)
    idx_i = _prep_edges(edge_index_u2i)
    zrows = jnp.zeros((ACC_ROWS - (NS - 1) * WR, D), jnp.float32)
    agg_user, agg_item = _sc_conv(h_i, h_u, idx_u, idx_i, zrows)
    return (agg_user, agg_item)


# speedup vs baseline: 1.2709x; 1.1888x over previous
import functools

import jax
import jax.numpy as jnp
from jax import lax
from jax.experimental import pallas as pl
from jax.experimental.pallas import tpu as pltpu
from jax.experimental.pallas import tpu_sc as plsc

N = 10000
D = 128
E = 320000
NC = 2
NS = 16
C = 128
NB = 3
NI = 2 * NB
U = NI
NCH = 159
EPT = NCH * C
ACC_ROWS = 10016
WR = 624


def _mm_body(x_ref, w_ref, o_ref):
    o_ref[...] = jnp.dot(x_ref[...], w_ref[...],
                         preferred_element_type=jnp.float32)


def _matmul(x, w):
    m = x.shape[0]
    bm = 2000
    return pl.pallas_call(
        _mm_body,
        grid=(m // bm,),
        in_specs=[pl.BlockSpec((bm, D), lambda i: (i, 0)),
                  pl.BlockSpec((D, D), lambda i: (0, 0))],
        out_specs=pl.BlockSpec((bm, D), lambda i: (i, 0)),
        out_shape=jax.ShapeDtypeStruct((m, D), jnp.float32),
    )(x, w)


def _prep_edges(edge_index):
    pad = NS * EPT - E
    src = jnp.concatenate(
        [edge_index[0].astype(jnp.int32),
         jnp.arange(pad, dtype=jnp.int32) % N]).reshape(NS, NCH, 1, C)
    dst = jnp.concatenate(
        [edge_index[1].astype(jnp.int32),
         N + (jnp.arange(pad, dtype=jnp.int32) % NS)]).reshape(NS, NCH, 1, C)
    return src, dst


def _sc_body(h0, h1, src0, dst0, src1, dst1, zrows,
             out0, out1,
             acc, bufs, ibs, sgs, scs, sis):
    c = lax.axis_index("c")
    s = lax.axis_index("s")

    @pl.when(s < NS - 1)
    def _():
        pltpu.sync_copy(zrows.at[pl.ds(0, WR)], acc.at[pl.ds(s * WR, WR)])

    @pl.when(s == NS - 1)
    def _():
        last = (NS - 1) * WR
        pltpu.sync_copy(zrows, acc.at[pl.ds(last, ACC_ROWS - last)])

    plsc.subcore_barrier()

    def run(h, src, dst, out):
        def idx_start(k, q):
            kc = jnp.minimum(k, NCH - 1)
            pltpu.make_async_copy(src.at[s, kc], ibs[q].at[pl.ds(0, 1)],
                                  sis[q]).start()
            pltpu.make_async_copy(dst.at[s, kc], ibs[q].at[pl.ds(1, 1)],
                                  sis[q]).start()

        def idx_wait(k, q):
            kc = jnp.minimum(k, NCH - 1)
            pltpu.make_async_copy(src.at[s, kc], ibs[q].at[pl.ds(0, 1)],
                                  sis[q]).wait()
            pltpu.make_async_copy(dst.at[s, kc], ibs[q].at[pl.ds(1, 1)],
                                  sis[q]).wait()

        def gather_start(p, q):
            pltpu.make_async_copy(h.at[ibs[q].at[0]], bufs[p], sgs[p]).start()

        def gather_wait(p, q):
            pltpu.make_async_copy(h.at[ibs[q].at[0]], bufs[p], sgs[p]).wait()

        def scatter_start(p, q):
            pltpu.async_copy(bufs[p], acc.at[ibs[q].at[1]], scs[p], add=True)

        def scatter_wait(p, q):
            pltpu.make_async_copy(bufs[p], acc.at[ibs[q].at[1]], scs[p]).wait()

        for k in range(NB):
            idx_start(k, k)
        for j in range(NB):
            idx_wait(j, j)
            gather_start(j, j)
            idx_start(j + NB, (j + NB) % NI)
            if j > 0:
                gather_wait(j - 1, j - 1)
                scatter_start(j - 1, j - 1)

        def body(g, carry):
            for r in range(U):
                j = NB + U * g + r
                p, q = (NB + r) % NB, (NB + r) % NI
                p1, q1 = (NB + r - 1) % NB, (NB + r - 1) % NI
                q3 = (NB + r + NB) % NI
                scatter_wait(p, q3)
                idx_wait(j, q)
                gather_start(p, q)
                idx_start(j + NB, q3)
                gather_wait(p1, q1)
                scatter_start(p1, q1)
            return carry

        lax.fori_loop(0, (NCH - NB) // U, body, 0)

        pL, qL = (NCH - 1) % NB, (NCH - 1) % NI
        gather_wait(pL, qL)
        scatter_start(pL, qL)
        for k in range(NCH - NB, NCH):
            scatter_wait(k % NB, k % NI)
        for k in range(NCH, NCH + NB):
            idx_wait(k, k % NI)

        plsc.subcore_barrier()

        @pl.when(s < NS - 1)
        def _():
            pltpu.sync_copy(acc.at[pl.ds(s * WR, WR)],
                            out.at[pl.ds(s * WR, WR)])

        @pl.when(s == NS - 1)
        def _():
            last = (NS - 1) * WR
            pltpu.sync_copy(acc.at[pl.ds(last, N - last)],
                            out.at[pl.ds(last, N - last)])

    @pl.when(c == 0)
    def _():
        run(h0, src0, dst0, out0)

    @pl.when(c == 1)
    def _():
        run(h1, src1, dst1, out1)


@functools.partial(
    pl.kernel,
    out_type=[jax.ShapeDtypeStruct((N, D), jnp.float32),
              jax.ShapeDtypeStruct((N, D), jnp.float32)],
    mesh=plsc.VectorSubcoreMesh(core_axis_name="c", subcore_axis_name="s",
                                num_cores=NC, num_subcores=NS),
    scratch_types=[
        pltpu.VMEM_SHARED((ACC_ROWS, D), jnp.float32),
        [pltpu.VMEM((C, D), jnp.float32) for _ in range(NB)],
        [pltpu.VMEM((2, C), jnp.int32) for _ in range(NI)],
        [pltpu.SemaphoreType.DMA for _ in range(NB)],
        [pltpu.SemaphoreType.DMA for _ in range(NB)],
        [pltpu.SemaphoreType.DMA for _ in range(NI)],
    ],
)
def _sc_conv(h0, h1, src0, dst0, src1, dst1, zrows, out0, out1,
             acc, bufs, ibs, sgs, scs, sis):
    _sc_body(h0, h1, src0, dst0, src1, dst1, zrows, out0, out1,
             acc, bufs, ibs, sgs, scs, sis)


def kernel(x_user, x_item, W_u2i, W_i2u, edge_index_u2i, edge_index_i2u):
    h_u = _matmul(x_user, W_u2i)
    h_i = _matmul(x_item, W_i2u)
    src_u, dst_u = _prep_edges(edge_index_i2u)
    src_i, dst_i = _prep_edges(edge_index_u2i)
    zrows = jnp.zeros((ACC_ROWS - (NS - 1) * WR, D), jnp.float32)
    agg_user, agg_item = _sc_conv(h_i, h_u, src_u, dst_u, src_i, dst_i, zrows)
    return (agg_user, agg_item)


# speedup vs baseline: 1.2728x; 1.0015x over previous
import functools

import jax
import jax.numpy as jnp
from jax import lax
from jax.experimental import pallas as pl
from jax.experimental.pallas import tpu as pltpu
from jax.experimental.pallas import tpu_sc as plsc

N = 10000
D = 128
E = 320000
NC = 2
NS = 16
C = 128
NB = 3
NI = 2 * NB
U = NI
NCH = 159
EPT = NCH * C
ACC_ROWS = 10016
WR = 624


def _mm_body(x_ref, w_ref, o_ref):
    o_ref[...] = jnp.dot(x_ref[...], w_ref[...],
                         preferred_element_type=jnp.float32)


def _matmul(x, w):
    m = x.shape[0]
    bm = 2000
    return pl.pallas_call(
        _mm_body,
        grid=(m // bm,),
        in_specs=[pl.BlockSpec((bm, D), lambda i: (i, 0)),
                  pl.BlockSpec((D, D), lambda i: (0, 0))],
        out_specs=pl.BlockSpec((bm, D), lambda i: (i, 0)),
        out_shape=jax.ShapeDtypeStruct((m, D), jnp.float32),
    )(x, w)


def _prep_edges(edge_index):
    pad = NS * EPT - E
    src = jnp.concatenate(
        [edge_index[0].astype(jnp.int32),
         jnp.arange(pad, dtype=jnp.int32) % N])
    dst = jnp.concatenate(
        [edge_index[1].astype(jnp.int32),
         N + (jnp.arange(pad, dtype=jnp.int32) % NS)])
    return src, dst


def _sc_body(h0, h1, src0, dst0, src1, dst1, zrows,
             out0, out1,
             acc, bufs, ibs, sgs, scs, sis):
    c = lax.axis_index("c")
    s = lax.axis_index("s")

    @pl.when(s < NS - 1)
    def _():
        pltpu.sync_copy(zrows.at[pl.ds(0, WR)], acc.at[pl.ds(s * WR, WR)])

    @pl.when(s == NS - 1)
    def _():
        last = (NS - 1) * WR
        pltpu.sync_copy(zrows, acc.at[pl.ds(last, ACC_ROWS - last)])

    plsc.subcore_barrier()

    def run(h, src, dst, out):
        def _idx_off(k):
            kc = jnp.minimum(k, NCH - 1)
            return pl.multiple_of((s * NCH + kc) * C, C)

        def idx_start(k, q):
            off = _idx_off(k)
            pltpu.make_async_copy(src.at[pl.ds(off, C)], ibs[q].at[0],
                                  sis[q]).start()
            pltpu.make_async_copy(dst.at[pl.ds(off, C)], ibs[q].at[1],
                                  sis[q]).start()

        def idx_wait(k, q):
            off = _idx_off(k)
            pltpu.make_async_copy(src.at[pl.ds(off, C)], ibs[q].at[0],
                                  sis[q]).wait()
            pltpu.make_async_copy(dst.at[pl.ds(off, C)], ibs[q].at[1],
                                  sis[q]).wait()

        def gather_start(p, q):
            pltpu.make_async_copy(h.at[ibs[q].at[0]], bufs[p], sgs[p]).start()

        def gather_wait(p, q):
            pltpu.make_async_copy(h.at[ibs[q].at[0]], bufs[p], sgs[p]).wait()

        def scatter_start(p, q):
            pltpu.async_copy(bufs[p], acc.at[ibs[q].at[1]], scs[p], add=True)

        def scatter_wait(p, q):
            pltpu.make_async_copy(bufs[p], acc.at[ibs[q].at[1]], scs[p]).wait()

        for k in range(NB):
            idx_start(k, k)
        for j in range(NB):
            idx_wait(j, j)
            gather_start(j, j)
            idx_start(j + NB, (j + NB) % NI)
            if j > 0:
                gather_wait(j - 1, j - 1)
                scatter_start(j - 1, j - 1)

        def body(g, carry):
            for r in range(U):
                j = NB + U * g + r
                p, q = (NB + r) % NB, (NB + r) % NI
                p1, q1 = (NB + r - 1) % NB, (NB + r - 1) % NI
                q3 = (NB + r + NB) % NI
                scatter_wait(p, q3)
                idx_wait(j, q)
                gather_start(p, q)
                idx_start(j + NB, q3)
                gather_wait(p1, q1)
                scatter_start(p1, q1)
            return carry

        lax.fori_loop(0, (NCH - NB) // U, body, 0)

        pL, qL = (NCH - 1) % NB, (NCH - 1) % NI
        gather_wait(pL, qL)
        scatter_start(pL, qL)
        for k in range(NCH - NB, NCH):
            scatter_wait(k % NB, k % NI)
        for k in range(NCH, NCH + NB):
            idx_wait(k, k % NI)

        plsc.subcore_barrier()

        @pl.when(s < NS - 1)
        def _():
            pltpu.sync_copy(acc.at[pl.ds(s * WR, WR)],
                            out.at[pl.ds(s * WR, WR)])

        @pl.when(s == NS - 1)
        def _():
            last = (NS - 1) * WR
            pltpu.sync_copy(acc.at[pl.ds(last, N - last)],
                            out.at[pl.ds(last, N - last)])

    @pl.when(c == 0)
    def _():
        run(h0, src0, dst0, out0)

    @pl.when(c == 1)
    def _():
        run(h1, src1, dst1, out1)


@functools.partial(
    pl.kernel,
    out_type=[jax.ShapeDtypeStruct((N, D), jnp.float32),
              jax.ShapeDtypeStruct((N, D), jnp.float32)],
    mesh=plsc.VectorSubcoreMesh(core_axis_name="c", subcore_axis_name="s",
                                num_cores=NC, num_subcores=NS),
    scratch_types=[
        pltpu.VMEM_SHARED((ACC_ROWS, D), jnp.float32),
        [pltpu.VMEM((C, D), jnp.float32) for _ in range(NB)],
        [pltpu.VMEM((2, C), jnp.int32) for _ in range(NI)],
        [pltpu.SemaphoreType.DMA for _ in range(NB)],
        [pltpu.SemaphoreType.DMA for _ in range(NB)],
        [pltpu.SemaphoreType.DMA for _ in range(NI)],
    ],
)
def _sc_conv(h0, h1, src0, dst0, src1, dst1, zrows, out0, out1,
             acc, bufs, ibs, sgs, scs, sis):
    _sc_body(h0, h1, src0, dst0, src1, dst1, zrows, out0, out1,
             acc, bufs, ibs, sgs, scs, sis)


def kernel(x_user, x_item, W_u2i, W_i2u, edge_index_u2i, edge_index_i2u):
    h_u = _matmul(x_user, W_u2i)
    h_i = _matmul(x_item, W_i2u)
    src_u, dst_u = _prep_edges(edge_index_i2u)
    src_i, dst_i = _prep_edges(edge_index_u2i)
    zrows = jnp.zeros((ACC_ROWS - (NS - 1) * WR, D), jnp.float32)
    agg_user, agg_item = _sc_conv(h_i, h_u, src_u, dst_u, src_i, dst_i, zrows)
    return (agg_user, agg_item)
